# Initial kernel scaffold; baseline (speedup 1.0000x reference)
import jax
import jax.numpy as jnp
from jax.experimental import pallas as pl


def kernel(x, table):
    raise NotImplementedError("write your pallas kernel here")


# speedup vs baseline: 1.3069x; 1.3069x over previous
import functools

import jax
import jax.numpy as jnp
from jax import lax
from jax.experimental import pallas as pl
from jax.experimental.pallas import tpu as pltpu
from jax.experimental.pallas import tpu_sc as plsc

NUM_EMB = 1_000_000
D = 32
B = 4096 * 200
NC = 2
NS = 16
NW = NC * NS
CHUNK = 128
PER_W = B // NW
NCHUNK = PER_W // CHUNK

_mesh = plsc.VectorSubcoreMesh(core_axis_name="c", subcore_axis_name="s")


@functools.partial(
    pl.kernel,
    mesh=_mesh,
    out_type=jax.ShapeDtypeStruct((NW, NCHUNK, CHUNK, D), jnp.float32),
    scratch_types=[
        pltpu.VMEM((NCHUNK, CHUNK), jnp.int32),
        pltpu.VMEM((CHUNK, D), jnp.float32),
        pltpu.SemaphoreType.DMA,
    ],
    compiler_params=pltpu.CompilerParams(use_tc_tiling_on_sc=False),
)
def _emb_lookup(table_hbm, idx_hbm, out_hbm, idx_v, rows_v, gsem):
    wid = lax.axis_index("s") * NC + lax.axis_index("c")
    pltpu.sync_copy(idx_hbm.at[wid], idx_v)

    def body(j, carry):
        pltpu.async_copy(table_hbm.at[idx_v.at[j]], rows_v, gsem).wait()
        pltpu.sync_copy(rows_v, out_hbm.at[wid, j])
        return carry

    lax.fori_loop(0, NCHUNK, body, 0)


def kernel(x, table):
    idx = x.astype(jnp.int32).reshape(NW, NCHUNK, CHUNK)
    out = _emb_lookup(table, idx)
    return out.reshape(4096, 200, D)


# speedup vs baseline: 1.5012x; 1.1487x over previous
import functools

import jax
import jax.numpy as jnp
from jax import lax
from jax.experimental import pallas as pl
from jax.experimental.pallas import tpu as pltpu
from jax.experimental.pallas import tpu_sc as plsc

NUM_EMB = 1_000_000
D = 32
B = 4096 * 200
NC = 2
NS = 16
NW = NC * NS
CHUNK = 128
GRP = 5
NBUF = 4
PER_W = B // NW
NGRP = PER_W // (GRP * CHUNK)
OUTER = NGRP // NBUF

_mesh = plsc.VectorSubcoreMesh(core_axis_name="c", subcore_axis_name="s")


@functools.partial(
    pl.kernel,
    mesh=_mesh,
    out_type=jax.ShapeDtypeStruct((NW, NGRP, GRP * CHUNK, D), jnp.float32),
    scratch_types=(
        [pltpu.VMEM((NGRP, GRP * CHUNK), jnp.int32)]
        + [pltpu.VMEM((GRP * CHUNK, D), jnp.float32) for _ in range(NBUF)]
        + [pltpu.SemaphoreType.DMA for _ in range(2 * NBUF)]
    ),
    compiler_params=pltpu.CompilerParams(use_tc_tiling_on_sc=False),
)
def _emb_lookup(table_hbm, idx_hbm, out_hbm, idx_v, *bufs_sems):
    bufs = bufs_sems[:NBUF]
    gsem = bufs_sems[NBUF:2 * NBUF]
    wsem = bufs_sems[2 * NBUF:]
    wid = lax.axis_index("s") * NC + lax.axis_index("c")
    pltpu.sync_copy(idx_hbm.at[wid], idx_v)

    for b in range(NBUF):
        pltpu.async_copy(table_hbm.at[idx_v.at[b]], bufs[b], gsem[b])

    def body(o, carry):
        for b in range(NBUF):
            g = o * NBUF + b
            pltpu.make_async_copy(table_hbm.at[idx_v.at[g]], bufs[b], gsem[b]).wait()
            wcopy = pltpu.async_copy(bufs[b], out_hbm.at[wid, g], wsem[b])

            @pl.when(o < OUTER - 1)
            def _():
                wcopy.wait()
                pltpu.async_copy(
                    table_hbm.at[idx_v.at[g + NBUF]], bufs[b], gsem[b]
                )

        return carry

    lax.fori_loop(0, OUTER, body, 0)

    for b in range(NBUF):
        pltpu.make_async_copy(bufs[b], out_hbm.at[wid, NGRP - NBUF + b], wsem[b]).wait()


def kernel(x, table):
    idx = x.astype(jnp.int32).reshape(NW, NGRP, GRP * CHUNK)
    out = _emb_lookup(table, idx)
    return out.reshape(4096, 200, D)
